# baseline (speedup 1.0000x reference)
import jax
import jax.numpy as jnp
from jax.experimental import pallas as pl
from jax.experimental.pallas import tpu as pltpu
from jax.experimental.pallas import tpu_sc as plsc

VOCAB = 100000
HIDDEN = 128
BASIN = 64

_VOCAB_BLOCK = 4000
_GATHER_WINDOW = 256


def _proj_softmax_body(w_ref, tp_ref, out_ref):
    logits = jax.lax.dot_general(
        tp_ref[...], w_ref[...],
        (((1,), (1,)), ((), ())),
        preferred_element_type=jnp.float32,
    )
    m = jnp.max(logits, axis=-1, keepdims=True)
    e = jnp.exp(logits - m)
    sm = e / jnp.sum(e, axis=-1, keepdims=True)
    out_ref[:, :BASIN] = sm


def _project_softmax_table(token_params, basin_proj_w):
    grid = VOCAB // _VOCAB_BLOCK
    return pl.pallas_call(
        _proj_softmax_body,
        grid=(grid,),
        in_specs=[
            pl.BlockSpec((BASIN, HIDDEN), lambda i: (0, 0)),
            pl.BlockSpec((_VOCAB_BLOCK, HIDDEN), lambda i: (i, 0)),
        ],
        out_specs=pl.BlockSpec((_VOCAB_BLOCK, 2 * BASIN), lambda i: (i, 0)),
        out_shape=jax.ShapeDtypeStruct((VOCAB, 2 * BASIN), jnp.float32),
    )(basin_proj_w, token_params)


def _sc_gather(table, flat_ids):
    num_indices = flat_ids.shape[0]
    row_w = table.shape[1]
    ids2d = flat_ids.reshape(1, num_indices)
    mesh = plsc.VectorSubcoreMesh(core_axis_name="core",
                                  subcore_axis_name="subcore")
    n_sub = 16
    n_workers = 2 * n_sub
    W = _GATHER_WINDOW
    n_win = num_indices // W
    per_worker = n_win // n_workers
    n_pairs = per_worker // 2

    @pl.kernel(
        out_type=jax.ShapeDtypeStruct((num_indices, row_w), jnp.float32),
        mesh=mesh,
        scratch_types=[
            pltpu.VMEM((W,), jnp.int32),
            pltpu.VMEM((W,), jnp.int32),
            pltpu.VMEM((W, row_w), jnp.float32),
            pltpu.VMEM((W, row_w), jnp.float32),
            pltpu.SemaphoreType.DMA,
            pltpu.SemaphoreType.DMA,
            pltpu.SemaphoreType.DMA,
            pltpu.SemaphoreType.DMA,
            pltpu.SemaphoreType.DMA,
            pltpu.SemaphoreType.DMA,
        ],
    )
    def gather_kernel(table_hbm, ids_hbm, out_hbm,
                      ids_a, ids_b, rows_a, rows_b,
                      isem_a, isem_b, gsem_a, gsem_b, osem_a, osem_b):
        core = jax.lax.axis_index("core")
        sub = jax.lax.axis_index("subcore")
        worker = core * n_sub + sub
        first = worker * per_worker

        def base(k):
            return (first + k) * W

        def ids_cp(k, buf, sem):
            return pltpu.make_async_copy(
                ids_hbm.at[0, pl.ds(base(k), W)], buf, sem)

        def gather_cp(buf_ids, buf_rows, sem):
            return pltpu.make_async_copy(
                table_hbm.at[buf_ids], buf_rows, sem)

        def out_cp(k, buf_rows, sem):
            return pltpu.make_async_copy(
                buf_rows, out_hbm.at[pl.ds(base(k), W), :], sem)

        ids_cp(0, ids_a, isem_a).start()
        ids_cp(1, ids_b, isem_b).start()

        @pl.loop(0, n_pairs)
        def _(p):
            k0 = 2 * p
            k1 = k0 + 1
            ids_cp(k0, ids_a, isem_a).wait()

            @pl.when(p >= 1)
            def _():
                out_cp(k0 - 2, rows_a, osem_a).wait()

            gather_cp(ids_a, rows_a, gsem_a).start()
            ids_cp(k1, ids_b, isem_b).wait()

            @pl.when(p >= 1)
            def _():
                out_cp(k1 - 2, rows_b, osem_b).wait()

            gather_cp(ids_b, rows_b, gsem_b).start()

            gather_cp(ids_a, rows_a, gsem_a).wait()

            @pl.when(p + 1 < n_pairs)
            def _():
                ids_cp(k0 + 2, ids_a, isem_a).start()

            out_cp(k0, rows_a, osem_a).start()
            gather_cp(ids_b, rows_b, gsem_b).wait()

            @pl.when(p + 1 < n_pairs)
            def _():
                ids_cp(k1 + 2, ids_b, isem_b).start()

            out_cp(k1, rows_b, osem_b).start()

        out_cp(per_worker - 2, rows_a, osem_a).wait()
        out_cp(per_worker - 1, rows_b, osem_b).wait()

    return gather_kernel(table, ids2d)


@jax.jit
def kernel(token_ids, token_params, basin_proj_w):
    B, T = token_ids.shape
    table = _project_softmax_table(token_params, basin_proj_w)
    flat = token_ids.reshape(B * T).astype(jnp.int32)
    n = B * T
    n_pad = ((n + _GATHER_WINDOW - 1) // _GATHER_WINDOW) * _GATHER_WINDOW
    if n_pad != n:
        flat = jnp.pad(flat, (0, n_pad - n))
    g = _sc_gather(table, flat)
    return g[:n, :BASIN].reshape(B, T, BASIN)


# speedup vs baseline: 1.0079x; 1.0079x over previous
import jax
import jax.numpy as jnp
from jax.experimental import pallas as pl
from jax.experimental.pallas import tpu as pltpu
from jax.experimental.pallas import tpu_sc as plsc

VOCAB = 100000
HIDDEN = 128
BASIN = 64

_VOCAB_BLOCK = 4000
_GATHER_WINDOW = 256


def _proj_softmax_body(w_ref, tp_ref, out_ref):
    logits = jax.lax.dot_general(
        tp_ref[...], w_ref[...],
        (((1,), (1,)), ((), ())),
        preferred_element_type=jnp.float32,
    )
    m = jnp.max(logits, axis=-1, keepdims=True)
    e = jnp.exp(logits - m)
    sm = e / jnp.sum(e, axis=-1, keepdims=True)
    out_ref[:, :BASIN] = sm


def _project_softmax_table(token_params, basin_proj_w):
    grid = VOCAB // _VOCAB_BLOCK
    return pl.pallas_call(
        _proj_softmax_body,
        grid=(grid,),
        in_specs=[
            pl.BlockSpec((BASIN, HIDDEN), lambda i: (0, 0)),
            pl.BlockSpec((_VOCAB_BLOCK, HIDDEN), lambda i: (i, 0)),
        ],
        out_specs=pl.BlockSpec((_VOCAB_BLOCK, 2 * BASIN), lambda i: (i, 0)),
        out_shape=jax.ShapeDtypeStruct((VOCAB, 2 * BASIN), jnp.float32),
    )(basin_proj_w, token_params)


def _sc_gather(table, flat_ids):
    num_indices = flat_ids.shape[0]
    row_w = table.shape[1]
    ids2d = flat_ids.reshape(1, num_indices)
    mesh = plsc.VectorSubcoreMesh(core_axis_name="core",
                                  subcore_axis_name="subcore")

    @pl.kernel(
        out_type=jax.ShapeDtypeStruct((num_indices, row_w), jnp.float32),
        mesh=mesh,
    )
    def gather_kernel(table_hbm, ids_hbm, out_hbm):
        def body(ids_vmem, out_vmem):
            pltpu.sync_copy(table_hbm.at[ids_vmem.at[0]], out_vmem)

        pltpu.emit_pipeline(
            body,
            grid=(num_indices // _GATHER_WINDOW,),
            in_specs=[pl.BlockSpec((1, _GATHER_WINDOW),
                                   index_map=lambda i: (0, i))],
            out_specs=[pl.BlockSpec((_GATHER_WINDOW, row_w),
                                    index_map=lambda i: (i, 0))],
            core_axis_name=("core", "subcore"),
            dimension_semantics=(pltpu.PARALLEL,),
        )(ids_hbm, out_hbm)

    return gather_kernel(table, ids2d)


@jax.jit
def kernel(token_ids, token_params, basin_proj_w):
    B, T = token_ids.shape
    table = _project_softmax_table(token_params, basin_proj_w)
    flat = token_ids.reshape(B * T).astype(jnp.int32)
    n = B * T
    n_pad = ((n + _GATHER_WINDOW - 1) // _GATHER_WINDOW) * _GATHER_WINDOW
    if n_pad != n:
        flat = jnp.pad(flat, (0, n_pad - n))
    g = _sc_gather(table, flat)
    return g[:n, :BASIN].reshape(B, T, BASIN)


# speedup vs baseline: 1.0241x; 1.0160x over previous
import jax
import jax.numpy as jnp
from jax.experimental import pallas as pl
from jax.experimental.pallas import tpu as pltpu
from jax.experimental.pallas import tpu_sc as plsc

VOCAB = 100000
HIDDEN = 128
BASIN = 64

_VOCAB_BLOCK = 10000
_GATHER_WINDOW = 256


def _proj_softmax_body(w_ref, tp_ref, out_ref):
    logits = jax.lax.dot_general(
        tp_ref[...], w_ref[...],
        (((1,), (1,)), ((), ())),
        preferred_element_type=jnp.float32,
    )
    m = jnp.max(logits, axis=-1, keepdims=True)
    e = jnp.exp(logits - m)
    sm = e / jnp.sum(e, axis=-1, keepdims=True)
    out_ref[:, :BASIN] = sm


def _project_softmax_table(token_params, basin_proj_w):
    grid = VOCAB // _VOCAB_BLOCK
    return pl.pallas_call(
        _proj_softmax_body,
        grid=(grid,),
        in_specs=[
            pl.BlockSpec((BASIN, HIDDEN), lambda i: (0, 0)),
            pl.BlockSpec((_VOCAB_BLOCK, HIDDEN), lambda i: (i, 0)),
        ],
        out_specs=pl.BlockSpec((_VOCAB_BLOCK, 2 * BASIN), lambda i: (i, 0)),
        out_shape=jax.ShapeDtypeStruct((VOCAB, 2 * BASIN), jnp.float32),
    )(basin_proj_w, token_params)


def _sc_gather(table, flat_ids):
    num_indices = flat_ids.shape[0]
    row_w = table.shape[1]
    ids2d = flat_ids.reshape(1, num_indices)
    mesh = plsc.VectorSubcoreMesh(core_axis_name="core",
                                  subcore_axis_name="subcore")

    @pl.kernel(
        out_type=jax.ShapeDtypeStruct((num_indices, row_w), jnp.float32),
        mesh=mesh,
    )
    def gather_kernel(table_hbm, ids_hbm, out_hbm):
        def body(ids_vmem, out_vmem):
            pltpu.sync_copy(table_hbm.at[ids_vmem.at[0]], out_vmem)

        pltpu.emit_pipeline(
            body,
            grid=(num_indices // _GATHER_WINDOW,),
            in_specs=[pl.BlockSpec((1, _GATHER_WINDOW),
                                   index_map=lambda i: (0, i))],
            out_specs=[pl.BlockSpec((_GATHER_WINDOW, row_w),
                                    index_map=lambda i: (i, 0))],
            core_axis_name=("core", "subcore"),
            dimension_semantics=(pltpu.PARALLEL,),
        )(ids_hbm, out_hbm)

    return gather_kernel(table, ids2d)


@jax.jit
def kernel(token_ids, token_params, basin_proj_w):
    B, T = token_ids.shape
    table = _project_softmax_table(token_params, basin_proj_w)
    flat = token_ids.reshape(B * T).astype(jnp.int32)
    n = B * T
    n_pad = ((n + _GATHER_WINDOW - 1) // _GATHER_WINDOW) * _GATHER_WINDOW
    if n_pad != n:
        flat = jnp.pad(flat, (0, n_pad - n))
    g = _sc_gather(table, flat)
    return g[:n, :BASIN].reshape(B, T, BASIN)
